# baseline (speedup 1.0000x reference)
import functools

import jax
import jax.numpy as jnp
from jax import lax
from jax.experimental import pallas as pl
from jax.experimental.pallas import tpu as pltpu
from jax.experimental.pallas import tpu_sc as plsc

_VOCAB = 1000000
_EMBED_DIM = 32
_BATCH = 16384
_PACK = 128 // _EMBED_DIM

_NC = 2
_NS = 16
_NW = _NC * _NS
_B_PER_W = _BATCH // _NW
_CHUNK = 128
_NCHUNK = _B_PER_W // _CHUNK
_GROUPS = _B_PER_W // 16


def _gather_body(sup_hbm, off_hbm, table_hbm, out_hbm, sup_v, off_v, buf_v,
                 out_v, sem):
    wid = lax.axis_index("s") * _NC + lax.axis_index("c")
    base = wid * _B_PER_W
    pltpu.sync_copy(sup_hbm.at[pl.ds(base, _B_PER_W)], sup_v)
    pltpu.sync_copy(off_hbm.at[pl.ds(base, _B_PER_W)], off_v)
    copies = []
    for j in range(_NCHUNK):
        copies.append(
            pltpu.async_copy(
                table_hbm.at[sup_v.at[pl.ds(j * _CHUNK, _CHUNK)]],
                buf_v.at[pl.ds(j * _CHUNK, _CHUNK)],
                sem,
            )
        )
    for c in copies:
        c.wait()

    iota = lax.iota(jnp.int32, 16)

    def group(g, carry):
        rows = g * 16 + iota
        colbase = off_v[pl.ds(g * 16, 16)]
        orow = rows >> 2
        ocolbase = (rows & 3) * _EMBED_DIM
        for o in range(_EMBED_DIM):
            v = plsc.load_gather(buf_v, [rows, colbase + o])
            plsc.store_scatter(out_v, [orow, ocolbase + o], v)
        return carry

    lax.fori_loop(0, _GROUPS, group, 0)
    pltpu.sync_copy(out_v, out_hbm.at[pl.ds(wid * (_B_PER_W // _PACK),
                                            _B_PER_W // _PACK)])


@jax.jit
def _gather(sup, off, table2):
    mesh = plsc.VectorSubcoreMesh(core_axis_name="c", subcore_axis_name="s")
    run = functools.partial(
        pl.kernel,
        out_type=jax.ShapeDtypeStruct((_BATCH // _PACK, 128), jnp.float32),
        mesh=mesh,
        scratch_types=[
            pltpu.VMEM((_B_PER_W,), jnp.int32),
            pltpu.VMEM((_B_PER_W,), jnp.int32),
            pltpu.VMEM((_B_PER_W, 128), jnp.float32),
            pltpu.VMEM((_B_PER_W // _PACK, 128), jnp.float32),
            pltpu.SemaphoreType.DMA,
        ],
        compiler_params=pltpu.CompilerParams(needs_layout_passes=False),
    )(_gather_body)
    return run(sup, off, table2)


def kernel(selector, kernel):
    idx = jnp.reshape(selector, (-1,)).astype(jnp.int32)
    sup = idx >> 2
    off = (idx & 3) * _EMBED_DIM
    table2 = jnp.reshape(kernel, (_VOCAB // _PACK, 128))
    out2 = _gather(sup, off, table2)
    return jnp.reshape(out2, (_BATCH, _EMBED_DIM))


# speedup vs baseline: 3.1874x; 3.1874x over previous
import functools

import jax
import jax.numpy as jnp
from jax import lax
from jax.experimental import pallas as pl
from jax.experimental.pallas import tpu as pltpu
from jax.experimental.pallas import tpu_sc as plsc

_VOCAB = 1000000
_EMBED_DIM = 32
_BATCH = 16384

_NC = 2
_NS = 16
_NW = _NC * _NS
_TCOLS = 7813
_COLS_PER_W = 244
_CHUNK_COLS = 8
_CHUNK = _CHUNK_COLS * 128
_NCHUNK = 31
_MAX_OFF = _VOCAB - _CHUNK - (_VOCAB - _CHUNK) % 128
_TAIL_OFF = 999936
_LIST = _BATCH + 16
_DUMP = _BATCH


def _stream_body(idx_hbm, table_hbm, out_hbm, idx_v, list_r, list_i,
                 crel, ci, chunk_v, stage_v, rowidx_v, sem):
    wid = lax.axis_index("s") * _NC + lax.axis_index("c")
    iota = lax.iota(jnp.int32, 16)
    cstart = wid * _COLS_PER_W + jnp.minimum(wid, 5)
    cend = cstart + _COLS_PER_W + jnp.where(wid < 5, 1, 0)
    lo = cstart * 128
    hi = jnp.minimum(cend * 128, _VOCAB)

    pltpu.sync_copy(idx_hbm, idx_v)

    def scan_step(v, cnt):
        r = idx_v[pl.ds(v * 16, 16)]
        m = (r >= lo) & (r < hi)
        plsc.store_compressed(list_r.at[pl.ds(cnt, 16)], r, mask=m)
        plsc.store_compressed(list_i.at[pl.ds(cnt, 16)], v * 16 + iota, mask=m)
        inc = jnp.max(plsc.all_reduce_population_count(m))
        return cnt + inc

    cnt = lax.fori_loop(0, _BATCH // 16, scan_step, jnp.int32(0))
    list_r[pl.ds(cnt, 16)] = jnp.full((16,), -1, jnp.int32)
    list_i[pl.ds(cnt, 16)] = _DUMP + iota
    ntrip = (cnt + 15) // 16

    def do_chunk(clo, width_cols, match_lo):
        pltpu.sync_copy(
            table_hbm.at[:, pl.ds(clo, width_cols * 128)],
            chunk_v.at[:, pl.ds(0, width_cols * 128)],
        )
        chi = clo + width_cols * 128

        def pair_step(w, ccnt):
            rv = list_r[pl.ds(w * 16, 16)]
            iv = list_i[pl.ds(w * 16, 16)]
            m = (rv >= match_lo) & (rv < chi)
            plsc.store_compressed(crel.at[pl.ds(ccnt, 16)], rv - clo, mask=m)
            plsc.store_compressed(ci.at[pl.ds(ccnt, 16)], iv, mask=m)
            return ccnt + jnp.max(plsc.all_reduce_population_count(m))

        ccnt = lax.fori_loop(0, ntrip, pair_step, jnp.int32(0))
        crel[pl.ds(ccnt, 16)] = jnp.zeros((16,), jnp.int32)
        ci[pl.ds(ccnt, 16)] = _DUMP + iota

        def wave_step(w, carry):
            rel = crel[pl.ds(w * 16, 16)]
            rowidx_v[...] = ci[pl.ds(w * 16, 16)]
            for d in range(_EMBED_DIM):
                vals = plsc.load_gather(
                    chunk_v, [jnp.full((16,), d, jnp.int32), rel])
                plsc.store_scatter(
                    stage_v, [iota, jnp.full((16,), d, jnp.int32)], vals)
            pltpu.async_copy(stage_v, out_hbm.at[rowidx_v], sem).wait()
            return carry

        lax.fori_loop(0, (ccnt + 15) // 16, wave_step, jnp.int32(0))

    def chunk_step(k, carry):
        clo = pl.multiple_of(
            jnp.minimum((cstart + k * _CHUNK_COLS) * 128, _MAX_OFF), 128)
        do_chunk(clo, _CHUNK_COLS, jnp.maximum(clo, lo))
        return carry

    lax.fori_loop(0, _NCHUNK, chunk_step, jnp.int32(0))
    tail = pl.multiple_of(wid * 0 + _TAIL_OFF, 128)
    do_chunk(tail, 1, jnp.maximum(jnp.int32(_TAIL_OFF), lo))


@jax.jit
def _stream_gather(idx, table_t):
    mesh = plsc.VectorSubcoreMesh(core_axis_name="c", subcore_axis_name="s")
    run = functools.partial(
        pl.kernel,
        out_type=jax.ShapeDtypeStruct((_BATCH + 32, 128), jnp.float32),
        mesh=mesh,
        scratch_types=[
            pltpu.VMEM((_BATCH,), jnp.int32),
            pltpu.VMEM((_LIST,), jnp.int32),
            pltpu.VMEM((_LIST,), jnp.int32),
            pltpu.VMEM((_LIST,), jnp.int32),
            pltpu.VMEM((_LIST,), jnp.int32),
            pltpu.VMEM((_EMBED_DIM, _CHUNK), jnp.float32),
            pltpu.VMEM((16, 128), jnp.float32),
            pltpu.VMEM((16,), jnp.int32),
            pltpu.SemaphoreType.DMA,
        ],
        compiler_params=pltpu.CompilerParams(
            needs_layout_passes=False, disable_bounds_checks=True),
    )(_stream_body)
    return run(idx, table_t)


def kernel(selector, kernel):
    idx = jnp.reshape(selector, (-1,)).astype(jnp.int32)
    table_t = jnp.transpose(kernel)
    out_p = _stream_gather(idx, table_t)
    return out_p[:_BATCH, :_EMBED_DIM]
